# baseline (speedup 1.0000x reference)
import functools

import jax
import jax.numpy as jnp
from jax import lax
from jax.experimental import pallas as pl
from jax.experimental.pallas import tpu as pltpu
from jax.experimental.pallas import tpu_sc as plsc

VOCAB = 100000
EMBED = 128
BATCH = 16384
L = 16
NW = 32
BPW = BATCH // NW
CHUNK = 128
NCHUNK = BPW // CHUNK
GROUPS = CHUNK // L
KCH = EMBED // L


@functools.partial(
    pl.kernel,
    out_type=jax.ShapeDtypeStruct((NW, L), jnp.float32),
    mesh=plsc.VectorSubcoreMesh(core_axis_name="c", subcore_axis_name="s"),
    compiler_params=pltpu.CompilerParams(needs_layout_passes=False),
    scratch_types=[
        pltpu.VMEM((BPW,), jnp.int32),
        pltpu.VMEM((BPW,), jnp.int32),
        pltpu.VMEM((BPW,), jnp.float32),
        pltpu.VMEM((BPW,), jnp.float32),
        pltpu.VMEM((CHUNK, EMBED), jnp.float32),
        pltpu.VMEM((CHUNK, EMBED), jnp.float32),
        pltpu.VMEM((CHUNK, EMBED), jnp.float32),
        pltpu.VMEM((CHUNK, EMBED), jnp.float32),
        pltpu.VMEM((CHUNK,), jnp.float32),
        pltpu.VMEM((CHUNK,), jnp.float32),
        pltpu.VMEM((CHUNK,), jnp.float32),
        pltpu.VMEM((CHUNK,), jnp.float32),
        pltpu.VMEM((L * (L + 1),), jnp.float32),
        pltpu.VMEM((L,), jnp.float32),
        pltpu.SemaphoreType.DMA,
        pltpu.SemaphoreType.DMA,
    ],
)
def _glove_partials(cw_hbm, tw_hbm, x_hbm, wt_hbm, wc_hbm, wo_hbm, vb_hbm,
                    ub_hbm, out_hbm, idxc_v, idxt_v, x_v, wt_v,
                    crow0_v, crow1_v, trow0_v, trow1_v,
                    cb0_v, cb1_v, tb0_v, tb1_v, st_v, acc_v, sem0, sem1):
    wid = lax.axis_index("s") * 2 + lax.axis_index("c")
    base = wid * BPW

    pltpu.sync_copy(cw_hbm.at[pl.ds(base, BPW)], idxc_v)
    pltpu.sync_copy(tw_hbm.at[pl.ds(base, BPW)], idxt_v)
    pltpu.sync_copy(x_hbm.at[pl.ds(base, BPW)], x_v)
    pltpu.sync_copy(wt_hbm.at[pl.ds(base, BPW)], wt_v)

    crow = (crow0_v, crow1_v)
    trow = (trow0_v, trow1_v)
    cb = (cb0_v, cb1_v)
    tb = (tb0_v, tb1_v)
    sems = (sem0, sem1)

    def fire(ci):
        par = ci % 2
        co = ci * CHUNK
        return [
            pltpu.async_copy(wc_hbm.at[idxc_v.at[pl.ds(co, CHUNK)]],
                             crow[par], sems[par]),
            pltpu.async_copy(wo_hbm.at[idxt_v.at[pl.ds(co, CHUNK)]],
                             trow[par], sems[par]),
            pltpu.async_copy(vb_hbm.at[idxc_v.at[pl.ds(co, CHUNK)]],
                             cb[par], sems[par]),
            pltpu.async_copy(ub_hbm.at[idxt_v.at[pl.ds(co, CHUNK)]],
                             tb[par], sems[par]),
        ]

    lane = lax.iota(jnp.int32, L)
    acc = jnp.zeros((L,), jnp.float32)
    pend = fire(0)
    for ci in range(NCHUNK):
        par = ci % 2
        co = ci * CHUNK
        nxt = fire(ci + 1) if ci + 1 < NCHUNK else None
        for cp in pend:
            cp.wait()
        pend = nxt
        crow_v, trow_v, cb_v, tb_v = crow[par], trow[par], cb[par], tb[par]

        def group_body(g, acc, crow_v=crow_v, trow_v=trow_v, cb_v=cb_v,
                       tb_v=tb_v, co=co):
            for b in range(L):
                item = g * L + b
                prods = [crow_v[item, pl.ds(k * L, L)] * trow_v[item, pl.ds(k * L, L)]
                         for k in range(KCH)]
                while len(prods) > 1:
                    prods = [prods[i] + prods[i + 1] for i in range(0, len(prods), 2)]
                st_v[pl.ds(b * (L + 1), L)] = prods[0]
            row = lane * (L + 1)
            cols = [plsc.load_gather(st_v, [row + j]) for j in range(L)]
            while len(cols) > 1:
                cols = [cols[i] + cols[i + 1] for i in range(0, len(cols), 2)]
            d = cols[0]
            gb = g * L
            r = d + cb_v[pl.ds(gb, L)] + tb_v[pl.ds(gb, L)] - x_v[pl.ds(co + gb, L)]
            return acc + wt_v[pl.ds(co + gb, L)] * r * r

        acc = lax.fori_loop(0, GROUPS, group_body, acc)

    acc_v[...] = acc
    pltpu.sync_copy(acc_v, out_hbm.at[wid])


def _sum_body(x_ref, o_ref):
    o_ref[...] = jnp.sum(x_ref[...]).reshape(1, 1)


def _sum_partials(p):
    return pl.pallas_call(
        _sum_body,
        out_shape=jax.ShapeDtypeStruct((1, 1), jnp.float32),
    )(p)[0, 0]


def kernel(center_words, target_words, coocs, weighting, W_center, W_outside,
           v_bias, u_bias):
    cw = center_words.reshape(BATCH)
    tw = target_words.reshape(BATCH)
    x = coocs.reshape(BATCH)
    w = weighting.reshape(BATCH)
    vb = v_bias.reshape(VOCAB)
    ub = u_bias.reshape(VOCAB)
    partials = _glove_partials(cw, tw, x, w, W_center, W_outside, vb, ub)
    return _sum_partials(partials.reshape(NW * L // EMBED, EMBED))


# speedup vs baseline: 1.0306x; 1.0306x over previous
import functools

import jax
import jax.numpy as jnp
from jax import lax
from jax.experimental import pallas as pl
from jax.experimental.pallas import tpu as pltpu
from jax.experimental.pallas import tpu_sc as plsc

VOCAB = 100000
EMBED = 128
BATCH = 16384
L = 16
NW = 32
BPW = BATCH // NW
CHUNK = 128
NCHUNK = BPW // CHUNK
GROUPS = CHUNK // L
KCH = EMBED // L


@functools.partial(
    pl.kernel,
    out_type=jax.ShapeDtypeStruct((NW, L), jnp.float32),
    mesh=plsc.VectorSubcoreMesh(core_axis_name="c", subcore_axis_name="s"),
    compiler_params=pltpu.CompilerParams(needs_layout_passes=False),
    scratch_types=[
        pltpu.VMEM((BPW,), jnp.int32),
        pltpu.VMEM((BPW,), jnp.int32),
        pltpu.VMEM((BPW,), jnp.float32),
        pltpu.VMEM((BPW,), jnp.float32),
        pltpu.VMEM((CHUNK, EMBED), jnp.float32),
        pltpu.VMEM((CHUNK, EMBED), jnp.float32),
        pltpu.VMEM((CHUNK, EMBED), jnp.float32),
        pltpu.VMEM((CHUNK, EMBED), jnp.float32),
        pltpu.VMEM((CHUNK,), jnp.float32),
        pltpu.VMEM((CHUNK,), jnp.float32),
        pltpu.VMEM((CHUNK,), jnp.float32),
        pltpu.VMEM((CHUNK,), jnp.float32),
        pltpu.VMEM((L * (L + 1),), jnp.float32),
        pltpu.VMEM((L,), jnp.float32),
        pltpu.SemaphoreType.DMA,
        pltpu.SemaphoreType.DMA,
    ],
)
def _glove_partials(cw_hbm, tw_hbm, x_hbm, wt_hbm, wc_hbm, wo_hbm, vb_hbm,
                    ub_hbm, out_hbm, idxc_v, idxt_v, x_v, wt_v,
                    crow0_v, crow1_v, trow0_v, trow1_v,
                    cb0_v, cb1_v, tb0_v, tb1_v, st_v, acc_v, sem0, sem1):
    wid = lax.axis_index("s") * 2 + lax.axis_index("c")
    base = wid * BPW

    pltpu.sync_copy(cw_hbm.at[pl.ds(base, BPW)], idxc_v)
    pltpu.sync_copy(tw_hbm.at[pl.ds(base, BPW)], idxt_v)
    pltpu.sync_copy(x_hbm.at[pl.ds(base, BPW)], x_v)
    pltpu.sync_copy(wt_hbm.at[pl.ds(base, BPW)], wt_v)

    crow = (crow0_v, crow1_v)
    trow = (trow0_v, trow1_v)
    cb = (cb0_v, cb1_v)
    tb = (tb0_v, tb1_v)
    sems = (sem0, sem1)

    def fire(ci):
        par = ci % 2
        co = ci * CHUNK
        return [
            pltpu.async_copy(wc_hbm.at[idxc_v.at[pl.ds(co, CHUNK)]],
                             crow[par], sems[par]),
            pltpu.async_copy(wo_hbm.at[idxt_v.at[pl.ds(co, CHUNK)]],
                             trow[par], sems[par]),
            pltpu.async_copy(vb_hbm.at[idxc_v.at[pl.ds(co, CHUNK)]],
                             cb[par], sems[par]),
            pltpu.async_copy(ub_hbm.at[idxt_v.at[pl.ds(co, CHUNK)]],
                             tb[par], sems[par]),
        ]

    lane = lax.iota(jnp.int32, L)
    acc = jnp.zeros((L,), jnp.float32)
    pend = fire(0)
    for ci in range(NCHUNK):
        par = ci % 2
        co = ci * CHUNK
        nxt = fire(ci + 1) if ci + 1 < NCHUNK else None
        for cp in pend:
            cp.wait()
        pend = nxt
        crow_v, trow_v, cb_v, tb_v = crow[par], trow[par], cb[par], tb[par]

        def group_body(g, acc, crow_v=crow_v, trow_v=trow_v, cb_v=cb_v,
                       tb_v=tb_v, co=co):
            for b in range(L):
                item = g * L + b
                sprod = crow_v[item, pl.ds(0, L)] * trow_v[item, pl.ds(0, L)]
                for k in range(1, KCH):
                    sprod = sprod + (crow_v[item, pl.ds(k * L, L)] *
                                     trow_v[item, pl.ds(k * L, L)])
                st_v[pl.ds(b * (L + 1), L)] = sprod
            row = lane * (L + 1)
            d = plsc.load_gather(st_v, [row])
            for j in range(1, L):
                d = d + plsc.load_gather(st_v, [row + j])
            gb = g * L
            r = d + cb_v[pl.ds(gb, L)] + tb_v[pl.ds(gb, L)] - x_v[pl.ds(co + gb, L)]
            return acc + wt_v[pl.ds(co + gb, L)] * r * r

        acc = lax.fori_loop(0, GROUPS, group_body, acc)

    acc_v[...] = acc
    pltpu.sync_copy(acc_v, out_hbm.at[wid])


def _sum_body(x_ref, o_ref):
    o_ref[...] = jnp.sum(x_ref[...]).reshape(1, 1)


def _sum_partials(p):
    return pl.pallas_call(
        _sum_body,
        out_shape=jax.ShapeDtypeStruct((1, 1), jnp.float32),
    )(p)[0, 0]


def kernel(center_words, target_words, coocs, weighting, W_center, W_outside,
           v_bias, u_bias):
    cw = center_words.reshape(BATCH)
    tw = target_words.reshape(BATCH)
    x = coocs.reshape(BATCH)
    w = weighting.reshape(BATCH)
    vb = v_bias.reshape(VOCAB)
    ub = u_bias.reshape(VOCAB)
    partials = _glove_partials(cw, tw, x, w, W_center, W_outside, vb, ub)
    return _sum_partials(partials.reshape(NW * L // EMBED, EMBED))


# speedup vs baseline: 1.0687x; 1.0370x over previous
import functools

import jax
import jax.numpy as jnp
from jax import lax
from jax.experimental import pallas as pl
from jax.experimental.pallas import tpu as pltpu
from jax.experimental.pallas import tpu_sc as plsc

VOCAB = 100000
EMBED = 128
BATCH = 16384
L = 16
NW = 32
BPW = BATCH // NW
CHUNK = 128
NCHUNK = BPW // CHUNK
GROUPS = CHUNK // L
KCH = EMBED // L


@functools.partial(
    pl.kernel,
    out_type=jax.ShapeDtypeStruct((NW * L // EMBED, EMBED), jnp.float32),
    mesh=plsc.VectorSubcoreMesh(core_axis_name="c", subcore_axis_name="s"),
    compiler_params=pltpu.CompilerParams(needs_layout_passes=False,
                                         disable_bounds_checks=True),
    scratch_types=[
        pltpu.VMEM((BPW,), jnp.int32),
        pltpu.VMEM((BPW,), jnp.int32),
        pltpu.VMEM((BPW,), jnp.float32),
        pltpu.VMEM((BPW,), jnp.float32),
        pltpu.VMEM((CHUNK, EMBED), jnp.float32),
        pltpu.VMEM((CHUNK, EMBED), jnp.float32),
        pltpu.VMEM((CHUNK, EMBED), jnp.float32),
        pltpu.VMEM((CHUNK, EMBED), jnp.float32),
        pltpu.VMEM((CHUNK,), jnp.float32),
        pltpu.VMEM((CHUNK,), jnp.float32),
        pltpu.VMEM((CHUNK,), jnp.float32),
        pltpu.VMEM((CHUNK,), jnp.float32),
        pltpu.VMEM((L * (L + 1),), jnp.float32),
        pltpu.VMEM((L,), jnp.float32),
        pltpu.SemaphoreType.DMA,
        pltpu.SemaphoreType.DMA,
    ],
)
def _glove_partials(cw_hbm, tw_hbm, x_hbm, wt_hbm, wc_hbm, wo_hbm, vb_hbm,
                    ub_hbm, out_hbm, idxc_v, idxt_v, x_v, wt_v,
                    crow0_v, crow1_v, trow0_v, trow1_v,
                    cb0_v, cb1_v, tb0_v, tb1_v, st_v, acc_v, sem0, sem1):
    wid = lax.axis_index("s") * 2 + lax.axis_index("c")
    base = wid * BPW

    pltpu.sync_copy(cw_hbm.at[pl.ds(base, BPW)], idxc_v)
    pltpu.sync_copy(tw_hbm.at[pl.ds(base, BPW)], idxt_v)
    pltpu.sync_copy(x_hbm.at[pl.ds(base, BPW)], x_v)
    pltpu.sync_copy(wt_hbm.at[pl.ds(base, BPW)], wt_v)

    crow = (crow0_v, crow1_v)
    trow = (trow0_v, trow1_v)
    cb = (cb0_v, cb1_v)
    tb = (tb0_v, tb1_v)
    sems = (sem0, sem1)

    def fire(ci):
        par = ci % 2
        co = ci * CHUNK
        return [
            pltpu.async_copy(wc_hbm.at[idxc_v.at[pl.ds(co, CHUNK)]],
                             crow[par], sems[par]),
            pltpu.async_copy(wo_hbm.at[idxt_v.at[pl.ds(co, CHUNK)]],
                             trow[par], sems[par]),
            pltpu.async_copy(vb_hbm.at[idxc_v.at[pl.ds(co, CHUNK)]],
                             cb[par], sems[par]),
            pltpu.async_copy(ub_hbm.at[idxt_v.at[pl.ds(co, CHUNK)]],
                             tb[par], sems[par]),
        ]

    lane = lax.iota(jnp.int32, L)
    acc = jnp.zeros((L,), jnp.float32)
    pend = fire(0)
    for ci in range(NCHUNK):
        par = ci % 2
        co = ci * CHUNK
        nxt = fire(ci + 1) if ci + 1 < NCHUNK else None
        for cp in pend:
            cp.wait()
        pend = nxt
        crow_v, trow_v, cb_v, tb_v = crow[par], trow[par], cb[par], tb[par]

        def group_body(g, acc, crow_v=crow_v, trow_v=trow_v, cb_v=cb_v,
                       tb_v=tb_v, co=co):
            for b in range(L):
                item = g * L + b
                sprod = crow_v[item, pl.ds(0, L)] * trow_v[item, pl.ds(0, L)]
                for k in range(1, KCH):
                    sprod = sprod + (crow_v[item, pl.ds(k * L, L)] *
                                     trow_v[item, pl.ds(k * L, L)])
                st_v[pl.ds(b * (L + 1), L)] = sprod
            row = lane * (L + 1)
            d = plsc.load_gather(st_v, [row])
            for j in range(1, L):
                d = d + plsc.load_gather(st_v, [row + j])
            gb = g * L
            r = d + cb_v[pl.ds(gb, L)] + tb_v[pl.ds(gb, L)] - x_v[pl.ds(co + gb, L)]
            return acc + wt_v[pl.ds(co + gb, L)] * r * r

        acc = lax.fori_loop(0, GROUPS, group_body, acc)

    acc_v[...] = acc
    pltpu.sync_copy(acc_v, out_hbm.at[wid // (EMBED // L),
                                      pl.ds((wid % (EMBED // L)) * L, L)])


def _sum_body(x_ref, o_ref):
    o_ref[...] = jnp.sum(x_ref[...]).reshape(1, 1)


def _sum_partials(p):
    return pl.pallas_call(
        _sum_body,
        out_shape=jax.ShapeDtypeStruct((1, 1), jnp.float32),
    )(p)[0, 0]


def kernel(center_words, target_words, coocs, weighting, W_center, W_outside,
           v_bias, u_bias):
    cw = center_words.reshape(BATCH)
    tw = target_words.reshape(BATCH)
    x = coocs.reshape(BATCH)
    w = weighting.reshape(BATCH)
    vb = v_bias.reshape(VOCAB)
    ub = u_bias.reshape(VOCAB)
    partials = _glove_partials(cw, tw, x, w, W_center, W_outside, vb, ub)
    return _sum_partials(partials)


# speedup vs baseline: 1.0947x; 1.0243x over previous
import functools

import jax
import jax.numpy as jnp
from jax import lax
from jax.experimental import pallas as pl
from jax.experimental.pallas import tpu as pltpu
from jax.experimental.pallas import tpu_sc as plsc

VOCAB = 100000
EMBED = 128
BATCH = 16384
L = 16
NW = 32
BPW = BATCH // NW
CHUNK = 128
NCHUNK = BPW // CHUNK
GROUPS = CHUNK // L
KCH = EMBED // L


@functools.partial(
    pl.kernel,
    out_type=jax.ShapeDtypeStruct((NW * L // EMBED, EMBED), jnp.float32),
    mesh=plsc.VectorSubcoreMesh(core_axis_name="c", subcore_axis_name="s"),
    compiler_params=pltpu.CompilerParams(needs_layout_passes=False,
                                         disable_bounds_checks=True),
    scratch_types=[
        pltpu.VMEM((BPW,), jnp.int32),
        pltpu.VMEM((BPW,), jnp.int32),
        pltpu.VMEM((BPW,), jnp.float32),
        pltpu.VMEM((BPW,), jnp.float32),
        pltpu.VMEM((CHUNK, EMBED), jnp.float32),
        pltpu.VMEM((CHUNK, EMBED), jnp.float32),
        pltpu.VMEM((CHUNK, EMBED), jnp.float32),
        pltpu.VMEM((CHUNK, EMBED), jnp.float32),
        pltpu.VMEM((CHUNK,), jnp.float32),
        pltpu.VMEM((CHUNK,), jnp.float32),
        pltpu.VMEM((CHUNK,), jnp.float32),
        pltpu.VMEM((CHUNK,), jnp.float32),
        pltpu.VMEM((L * (L + 1),), jnp.float32),
        pltpu.VMEM((L,), jnp.float32),
        pltpu.SemaphoreType.DMA,
        pltpu.SemaphoreType.DMA,
    ],
)
def _glove_partials(cw_hbm, tw_hbm, x_hbm, wt_hbm, wc_hbm, wo_hbm, vb_hbm,
                    ub_hbm, out_hbm, idxc_v, idxt_v, x_v, wt_v,
                    crow0_v, crow1_v, trow0_v, trow1_v,
                    cb0_v, cb1_v, tb0_v, tb1_v, st_v, acc_v, sem0, sem1):
    wid = lax.axis_index("s") * 2 + lax.axis_index("c")
    base = wid * BPW

    pltpu.sync_copy(cw_hbm.at[pl.ds(base, BPW)], idxc_v)
    pltpu.sync_copy(tw_hbm.at[pl.ds(base, BPW)], idxt_v)
    pltpu.sync_copy(x_hbm.at[pl.ds(base, BPW)], x_v)
    pltpu.sync_copy(wt_hbm.at[pl.ds(base, BPW)], wt_v)

    crow = (crow0_v, crow1_v)
    trow = (trow0_v, trow1_v)
    cb = (cb0_v, cb1_v)
    tb = (tb0_v, tb1_v)
    sems = (sem0, sem1)

    def fire(ci):
        par = ci % 2
        co = ci * CHUNK
        return [
            pltpu.async_copy(wc_hbm.at[idxc_v.at[pl.ds(co, CHUNK)]],
                             crow[par], sems[par]),
            pltpu.async_copy(wo_hbm.at[idxt_v.at[pl.ds(co, CHUNK)]],
                             trow[par], sems[par]),
            pltpu.async_copy(vb_hbm.at[0].at[idxc_v.at[pl.ds(co, CHUNK)]],
                             cb[par], sems[par]),
            pltpu.async_copy(ub_hbm.at[0].at[idxt_v.at[pl.ds(co, CHUNK)]],
                             tb[par], sems[par]),
        ]

    lane = lax.iota(jnp.int32, L)
    zcol = jnp.zeros((L,), jnp.int32)
    acc = jnp.zeros((L,), jnp.float32)
    pend = fire(0)
    for ci in range(NCHUNK):
        par = ci % 2
        co = ci * CHUNK
        nxt = fire(ci + 1) if ci + 1 < NCHUNK else None
        for cp in pend:
            cp.wait()
        pend = nxt
        crow_v, trow_v, cb_v, tb_v = crow[par], trow[par], cb[par], tb[par]

        def group_body(g, acc, crow_v=crow_v, trow_v=trow_v, cb_v=cb_v,
                       tb_v=tb_v, co=co):
            for b in range(L):
                item = g * L + b
                sprod = crow_v[item, pl.ds(0, L)] * trow_v[item, pl.ds(0, L)]
                for k in range(1, KCH):
                    sprod = sprod + (crow_v[item, pl.ds(k * L, L)] *
                                     trow_v[item, pl.ds(k * L, L)])
                st_v[pl.ds(b * (L + 1), L)] = sprod
            row = lane * (L + 1)
            d = plsc.load_gather(st_v, [row])
            for j in range(1, L):
                d = d + plsc.load_gather(st_v, [row + j])
            gb = g * L
            r = d + cb_v[pl.ds(gb, L)] + tb_v[pl.ds(gb, L)] - x_v[pl.ds(co + gb, L)]
            return acc + wt_v[pl.ds(co + gb, L)] * r * r

        acc = lax.fori_loop(0, GROUPS, group_body, acc)

    acc_v[...] = acc
    pltpu.sync_copy(acc_v, out_hbm.at[wid // (EMBED // L),
                                      pl.ds((wid % (EMBED // L)) * L, L)])


def _sum_body(x_ref, o_ref):
    o_ref[...] = jnp.sum(x_ref[...]).reshape(1, 1)


def _sum_partials(p):
    return pl.pallas_call(
        _sum_body,
        out_shape=jax.ShapeDtypeStruct((1, 1), jnp.float32),
    )(p)[0, 0]


def kernel(center_words, target_words, coocs, weighting, W_center, W_outside,
           v_bias, u_bias):
    cw = center_words.reshape(BATCH)
    tw = target_words.reshape(BATCH)
    x = coocs.reshape(BATCH)
    w = weighting.reshape(BATCH)
    partials = _glove_partials(cw, tw, x, w, W_center, W_outside,
                               v_bias.T, u_bias.T)
    return _sum_partials(partials)


# speedup vs baseline: 1.1513x; 1.0518x over previous
import functools

import jax
import jax.numpy as jnp
from jax import lax
from jax.experimental import pallas as pl
from jax.experimental.pallas import tpu as pltpu
from jax.experimental.pallas import tpu_sc as plsc

VOCAB = 100000
EMBED = 128
BATCH = 16384
L = 16
NW = 32
BPW = BATCH // NW
CHUNK = 128
NCHUNK = BPW // CHUNK
GROUPS = CHUNK // L
KCH = EMBED // L


@functools.partial(
    pl.kernel,
    out_type=jax.ShapeDtypeStruct((NW * L // EMBED, EMBED), jnp.float32),
    mesh=plsc.VectorSubcoreMesh(core_axis_name="c", subcore_axis_name="s"),
    compiler_params=pltpu.CompilerParams(needs_layout_passes=False,
                                         disable_bounds_checks=True),
    scratch_types=[
        pltpu.VMEM((BPW,), jnp.int32),
        pltpu.VMEM((BPW,), jnp.int32),
        pltpu.VMEM((BPW,), jnp.float32),
        pltpu.VMEM((BPW,), jnp.float32),
        pltpu.VMEM((CHUNK, EMBED), jnp.float32),
        pltpu.VMEM((CHUNK, EMBED), jnp.float32),
        pltpu.VMEM((CHUNK, EMBED), jnp.float32),
        pltpu.VMEM((CHUNK, EMBED), jnp.float32),
        pltpu.VMEM((CHUNK,), jnp.float32),
        pltpu.VMEM((CHUNK,), jnp.float32),
        pltpu.VMEM((CHUNK,), jnp.float32),
        pltpu.VMEM((CHUNK,), jnp.float32),
        pltpu.VMEM((L * (L + 1),), jnp.float32),
        pltpu.VMEM((L,), jnp.float32),
        pltpu.SemaphoreType.DMA,
        pltpu.SemaphoreType.DMA,
    ],
)
def _glove_partials(cw_hbm, tw_hbm, x_hbm, wt_hbm, wc_hbm, wo_hbm, vb_hbm,
                    ub_hbm, out_hbm, idxc_v, idxt_v, x_v, wt_v,
                    crow0_v, crow1_v, trow0_v, trow1_v,
                    cb0_v, cb1_v, tb0_v, tb1_v, st_v, acc_v, sem0, sem1):
    wid = lax.axis_index("s") * 2 + lax.axis_index("c")
    base = wid * BPW

    pltpu.sync_copy(cw_hbm.at[pl.ds(base, BPW)], idxc_v)
    pltpu.sync_copy(tw_hbm.at[pl.ds(base, BPW)], idxt_v)
    pltpu.sync_copy(x_hbm.at[pl.ds(base, BPW)], x_v)
    pltpu.sync_copy(wt_hbm.at[pl.ds(base, BPW)], wt_v)

    crow = (crow0_v, crow1_v)
    trow = (trow0_v, trow1_v)
    cb = (cb0_v, cb1_v)
    tb = (tb0_v, tb1_v)
    sems = (sem0, sem1)

    def fire(ci, par):
        co = ci * CHUNK
        return [
            pltpu.async_copy(wc_hbm.at[idxc_v.at[pl.ds(co, CHUNK)]],
                             crow[par], sems[par]),
            pltpu.async_copy(wo_hbm.at[idxt_v.at[pl.ds(co, CHUNK)]],
                             trow[par], sems[par]),
            pltpu.async_copy(vb_hbm.at[0].at[idxc_v.at[pl.ds(co, CHUNK)]],
                             cb[par], sems[par]),
            pltpu.async_copy(ub_hbm.at[0].at[idxt_v.at[pl.ds(co, CHUNK)]],
                             tb[par], sems[par]),
        ]

    def wait(ci, par):
        co = ci * CHUNK
        pltpu.make_async_copy(wc_hbm.at[idxc_v.at[pl.ds(co, CHUNK)]],
                              crow[par], sems[par]).wait()
        pltpu.make_async_copy(wo_hbm.at[idxt_v.at[pl.ds(co, CHUNK)]],
                              trow[par], sems[par]).wait()
        pltpu.make_async_copy(vb_hbm.at[0].at[idxc_v.at[pl.ds(co, CHUNK)]],
                              cb[par], sems[par]).wait()
        pltpu.make_async_copy(ub_hbm.at[0].at[idxt_v.at[pl.ds(co, CHUNK)]],
                              tb[par], sems[par]).wait()

    lane = lax.iota(jnp.int32, L)
    acc0 = jnp.zeros((L,), jnp.float32)

    def compute(co, par, acc):
        crow_v, trow_v, cb_v, tb_v = crow[par], trow[par], cb[par], tb[par]

        def group_body(g, acc):
            for b in range(L):
                item = g * L + b
                sprod = crow_v[item, pl.ds(0, L)] * trow_v[item, pl.ds(0, L)]
                for k in range(1, KCH):
                    sprod = sprod + (crow_v[item, pl.ds(k * L, L)] *
                                     trow_v[item, pl.ds(k * L, L)])
                st_v[pl.ds(b * (L + 1), L)] = sprod
            row = lane * (L + 1)
            d = plsc.load_gather(st_v, [row])
            for j in range(1, L):
                d = d + plsc.load_gather(st_v, [row + j])
            gb = g * L
            r = d + cb_v[pl.ds(gb, L)] + tb_v[pl.ds(gb, L)] - x_v[pl.ds(co + gb, L)]
            return acc + wt_v[pl.ds(co + gb, L)] * r * r

        return lax.fori_loop(0, GROUPS, group_body, acc)

    fire(0, 0)

    def pair_body(i, acc):
        c0 = 2 * i
        h1 = fire(c0 + 1, 1)
        wait(c0, 0)
        acc = compute(c0 * CHUNK, 0, acc)

        @pl.when(i < NCHUNK // 2 - 1)
        def _():
            fire(c0 + 2, 0)

        for h in h1:
            h.wait()
        return compute((c0 + 1) * CHUNK, 1, acc)

    acc = lax.fori_loop(0, NCHUNK // 2, pair_body, acc0)

    acc_v[...] = acc
    pltpu.sync_copy(acc_v, out_hbm.at[wid // (EMBED // L),
                                      pl.ds((wid % (EMBED // L)) * L, L)])


def _sum_body(x_ref, o_ref):
    o_ref[...] = jnp.sum(x_ref[...]).reshape(1, 1)


def _sum_partials(p):
    return pl.pallas_call(
        _sum_body,
        out_shape=jax.ShapeDtypeStruct((1, 1), jnp.float32),
    )(p)[0, 0]


def kernel(center_words, target_words, coocs, weighting, W_center, W_outside,
           v_bias, u_bias):
    cw = center_words.reshape(BATCH)
    tw = target_words.reshape(BATCH)
    x = coocs.reshape(BATCH)
    w = weighting.reshape(BATCH)
    partials = _glove_partials(cw, tw, x, w, W_center, W_outside,
                               v_bias.T, u_bias.T)
    return _sum_partials(partials)


# speedup vs baseline: 1.1738x; 1.0196x over previous
import functools

import jax
import jax.numpy as jnp
from jax import lax
from jax.experimental import pallas as pl
from jax.experimental.pallas import tpu as pltpu
from jax.experimental.pallas import tpu_sc as plsc

VOCAB = 100000
EMBED = 128
BATCH = 16384
L = 16
NW = 32
BPW = BATCH // NW
CHUNK = 128
NCHUNK = BPW // CHUNK
GROUPS = CHUNK // L
KCH = EMBED // L


@functools.partial(
    pl.kernel,
    out_type=jax.ShapeDtypeStruct((NW * L // EMBED, EMBED), jnp.float32),
    mesh=plsc.VectorSubcoreMesh(core_axis_name="c", subcore_axis_name="s"),
    compiler_params=pltpu.CompilerParams(needs_layout_passes=False,
                                         disable_bounds_checks=True),
    scratch_types=[
        pltpu.VMEM((BPW,), jnp.int32),
        pltpu.VMEM((BPW,), jnp.int32),
        pltpu.VMEM((BPW,), jnp.float32),
        pltpu.VMEM((BPW,), jnp.float32),
        pltpu.VMEM((2, CHUNK, EMBED), jnp.float32),
        pltpu.VMEM((2, CHUNK, EMBED), jnp.float32),
        pltpu.VMEM((2, CHUNK), jnp.float32),
        pltpu.VMEM((2, CHUNK), jnp.float32),
        pltpu.VMEM((L * (L + 1),), jnp.float32),
        pltpu.VMEM((L,), jnp.float32),
        pltpu.SemaphoreType.DMA((2,)),
    ],
)
def _glove_partials(cw_hbm, tw_hbm, x_hbm, wt_hbm, wc_hbm, wo_hbm, vb_hbm,
                    ub_hbm, out_hbm, idxc_v, idxt_v, x_v, wt_v,
                    crow2_v, trow2_v, cb2_v, tb2_v, st_v, acc_v, sem2):
    wid = lax.axis_index("s") * 2 + lax.axis_index("c")
    base = wid * BPW

    pltpu.sync_copy(cw_hbm.at[pl.ds(base, BPW)], idxc_v)
    pltpu.sync_copy(tw_hbm.at[pl.ds(base, BPW)], idxt_v)
    pltpu.sync_copy(x_hbm.at[pl.ds(base, BPW)], x_v)
    pltpu.sync_copy(wt_hbm.at[pl.ds(base, BPW)], wt_v)

    def fire(ci):
        par = lax.rem(ci, 2)
        co = ci * CHUNK
        return [
            pltpu.async_copy(wc_hbm.at[idxc_v.at[pl.ds(co, CHUNK)]],
                             crow2_v.at[par], sem2.at[par]),
            pltpu.async_copy(wo_hbm.at[idxt_v.at[pl.ds(co, CHUNK)]],
                             trow2_v.at[par], sem2.at[par]),
            pltpu.async_copy(vb_hbm.at[0].at[idxc_v.at[pl.ds(co, CHUNK)]],
                             cb2_v.at[par], sem2.at[par]),
            pltpu.async_copy(ub_hbm.at[0].at[idxt_v.at[pl.ds(co, CHUNK)]],
                             tb2_v.at[par], sem2.at[par]),
        ]

    def wait(ci):
        par = lax.rem(ci, 2)
        co = ci * CHUNK
        pltpu.make_async_copy(wc_hbm.at[idxc_v.at[pl.ds(co, CHUNK)]],
                              crow2_v.at[par], sem2.at[par]).wait()
        pltpu.make_async_copy(wo_hbm.at[idxt_v.at[pl.ds(co, CHUNK)]],
                              trow2_v.at[par], sem2.at[par]).wait()
        pltpu.make_async_copy(vb_hbm.at[0].at[idxc_v.at[pl.ds(co, CHUNK)]],
                              cb2_v.at[par], sem2.at[par]).wait()
        pltpu.make_async_copy(ub_hbm.at[0].at[idxt_v.at[pl.ds(co, CHUNK)]],
                              tb2_v.at[par], sem2.at[par]).wait()

    lane = lax.iota(jnp.int32, L)
    acc0 = jnp.zeros((L,), jnp.float32)

    fire(0)

    def chunk_body(ci, acc):
        par = lax.rem(ci, 2)
        co = ci * CHUNK

        @pl.when(ci < NCHUNK - 1)
        def _():
            fire(ci + 1)

        wait(ci)

        def group_body(g, acc):
            for b in range(L):
                item = g * L + b
                sprod = (crow2_v[par, item, pl.ds(0, L)] *
                         trow2_v[par, item, pl.ds(0, L)])
                for k in range(1, KCH):
                    sprod = sprod + (crow2_v[par, item, pl.ds(k * L, L)] *
                                     trow2_v[par, item, pl.ds(k * L, L)])
                st_v[pl.ds(b * (L + 1), L)] = sprod
            row = lane * (L + 1)
            d = plsc.load_gather(st_v, [row])
            for j in range(1, L):
                d = d + plsc.load_gather(st_v, [row + j])
            gb = g * L
            r = (d + cb2_v[par, pl.ds(gb, L)] + tb2_v[par, pl.ds(gb, L)]
                 - x_v[pl.ds(co + gb, L)])
            return acc + wt_v[pl.ds(co + gb, L)] * r * r

        return lax.fori_loop(0, GROUPS, group_body, acc)

    acc = lax.fori_loop(0, NCHUNK, chunk_body, acc0)

    acc_v[...] = acc
    pltpu.sync_copy(acc_v, out_hbm.at[wid // (EMBED // L),
                                      pl.ds((wid % (EMBED // L)) * L, L)])


def _sum_body(x_ref, o_ref):
    o_ref[...] = jnp.sum(x_ref[...]).reshape(1, 1)


def _sum_partials(p):
    return pl.pallas_call(
        _sum_body,
        out_shape=jax.ShapeDtypeStruct((1, 1), jnp.float32),
    )(p)[0, 0]


def kernel(center_words, target_words, coocs, weighting, W_center, W_outside,
           v_bias, u_bias):
    cw = center_words.reshape(BATCH)
    tw = target_words.reshape(BATCH)
    x = coocs.reshape(BATCH)
    w = weighting.reshape(BATCH)
    partials = _glove_partials(cw, tw, x, w, W_center, W_outside,
                               v_bias.T, u_bias.T)
    return _sum_partials(partials)


# speedup vs baseline: 1.2211x; 1.0402x over previous
import functools

import jax
import jax.numpy as jnp
from jax import lax
from jax.experimental import pallas as pl
from jax.experimental.pallas import tpu as pltpu
from jax.experimental.pallas import tpu_sc as plsc

VOCAB = 100000
EMBED = 128
BATCH = 16384
L = 16
NW = 32
BPW = BATCH // NW
CHUNK = 128
NCHUNK = BPW // CHUNK
GROUPS = CHUNK // L
KCH = EMBED // L


@functools.partial(
    pl.kernel,
    out_type=jax.ShapeDtypeStruct((NW * L // EMBED, EMBED), jnp.float32),
    mesh=plsc.VectorSubcoreMesh(core_axis_name="c", subcore_axis_name="s"),
    compiler_params=pltpu.CompilerParams(needs_layout_passes=False,
                                         disable_bounds_checks=True),
    scratch_types=[
        pltpu.VMEM((BPW,), jnp.int32),
        pltpu.VMEM((BPW,), jnp.int32),
        pltpu.VMEM((BPW,), jnp.float32),
        pltpu.VMEM((BPW,), jnp.float32),
        pltpu.VMEM((2, CHUNK, EMBED), jnp.float32),
        pltpu.VMEM((2, CHUNK, EMBED), jnp.float32),
        pltpu.VMEM((2, CHUNK), jnp.float32),
        pltpu.VMEM((2, CHUNK), jnp.float32),
        pltpu.VMEM((2 * L * (L + 1),), jnp.float32),
        pltpu.VMEM((L,), jnp.float32),
        pltpu.SemaphoreType.DMA((2,)),
        pltpu.SemaphoreType.DMA,
    ],
)
def _glove_partials(cw_hbm, tw_hbm, x_hbm, wt_hbm, wc_hbm, wo_hbm, vb_hbm,
                    ub_hbm, out_hbm, idxc_v, idxt_v, x_v, wt_v,
                    crow2_v, trow2_v, cb2_v, tb2_v, st_v, acc_v, sem2, semp):
    wid = lax.axis_index("s") * 2 + lax.axis_index("c")
    base = wid * BPW

    pcopies = [
        pltpu.async_copy(cw_hbm.at[pl.ds(base, BPW)], idxc_v, semp),
        pltpu.async_copy(tw_hbm.at[pl.ds(base, BPW)], idxt_v, semp),
        pltpu.async_copy(x_hbm.at[pl.ds(base, BPW)], x_v, semp),
        pltpu.async_copy(wt_hbm.at[pl.ds(base, BPW)], wt_v, semp),
    ]
    for cp in pcopies:
        cp.wait()

    def fire(ci):
        par = lax.rem(ci, 2)
        co = ci * CHUNK
        return [
            pltpu.async_copy(wc_hbm.at[idxc_v.at[pl.ds(co, CHUNK)]],
                             crow2_v.at[par], sem2.at[par]),
            pltpu.async_copy(wo_hbm.at[idxt_v.at[pl.ds(co, CHUNK)]],
                             trow2_v.at[par], sem2.at[par]),
            pltpu.async_copy(vb_hbm.at[0].at[idxc_v.at[pl.ds(co, CHUNK)]],
                             cb2_v.at[par], sem2.at[par]),
            pltpu.async_copy(ub_hbm.at[0].at[idxt_v.at[pl.ds(co, CHUNK)]],
                             tb2_v.at[par], sem2.at[par]),
        ]

    def wait(ci):
        par = lax.rem(ci, 2)
        co = ci * CHUNK
        pltpu.make_async_copy(wc_hbm.at[idxc_v.at[pl.ds(co, CHUNK)]],
                              crow2_v.at[par], sem2.at[par]).wait()
        pltpu.make_async_copy(wo_hbm.at[idxt_v.at[pl.ds(co, CHUNK)]],
                              trow2_v.at[par], sem2.at[par]).wait()
        pltpu.make_async_copy(vb_hbm.at[0].at[idxc_v.at[pl.ds(co, CHUNK)]],
                              cb2_v.at[par], sem2.at[par]).wait()
        pltpu.make_async_copy(ub_hbm.at[0].at[idxt_v.at[pl.ds(co, CHUNK)]],
                              tb2_v.at[par], sem2.at[par]).wait()

    lane = lax.iota(jnp.int32, L)
    acc0 = jnp.zeros((L,), jnp.float32)

    fire(0)

    def chunk_body(ci, acc):
        par = lax.rem(ci, 2)
        co = ci * CHUNK

        @pl.when(ci < NCHUNK - 1)
        def _():
            fire(ci + 1)

        wait(ci)

        def group_body(g, acc):
            so = lax.rem(g, 2) * (L * (L + 1))
            for b in range(L):
                item = g * L + b
                sprod = (crow2_v[par, item, pl.ds(0, L)] *
                         trow2_v[par, item, pl.ds(0, L)])
                for k in range(1, KCH):
                    sprod = sprod + (crow2_v[par, item, pl.ds(k * L, L)] *
                                     trow2_v[par, item, pl.ds(k * L, L)])
                st_v[pl.ds(so + b * (L + 1), L)] = sprod
            row = lane * (L + 1) + so
            d = plsc.load_gather(st_v, [row])
            for j in range(1, L):
                d = d + plsc.load_gather(st_v, [row + j])
            gb = g * L
            r = (d + cb2_v[par, pl.ds(gb, L)] + tb2_v[par, pl.ds(gb, L)]
                 - x_v[pl.ds(co + gb, L)])
            return acc + wt_v[pl.ds(co + gb, L)] * r * r

        return lax.fori_loop(0, GROUPS, group_body, acc)

    acc = lax.fori_loop(0, NCHUNK, chunk_body, acc0)

    acc_v[...] = acc
    pltpu.sync_copy(acc_v, out_hbm.at[wid // (EMBED // L),
                                      pl.ds((wid % (EMBED // L)) * L, L)])


def _sum_body(x_ref, o_ref):
    o_ref[...] = jnp.sum(x_ref[...]).reshape(1, 1)


def _sum_partials(p):
    return pl.pallas_call(
        _sum_body,
        out_shape=jax.ShapeDtypeStruct((1, 1), jnp.float32),
    )(p)[0, 0]


def kernel(center_words, target_words, coocs, weighting, W_center, W_outside,
           v_bias, u_bias):
    cw = center_words.reshape(BATCH)
    tw = target_words.reshape(BATCH)
    x = coocs.reshape(BATCH)
    w = weighting.reshape(BATCH)
    partials = _glove_partials(cw, tw, x, w, W_center, W_outside,
                               v_bias.T, u_bias.T)
    return _sum_partials(partials)
